# baseline (speedup 1.0000x reference)
import functools

import jax
import jax.numpy as jnp
from jax import lax
from jax.experimental import pallas as pl
from jax.experimental.pallas import tpu as pltpu
from jax.experimental.pallas import tpu_sc as plsc

NC = 2
NS = 16
NW = NC * NS


def _mesh():
    return plsc.VectorSubcoreMesh(
        core_axis_name="c", subcore_axis_name="s", num_cores=NC, num_subcores=NS
    )


def _make_sc_degree(n, ch, b, ring):
    rows = n // NS

    @functools.partial(
        pl.kernel,
        out_type=jax.ShapeDtypeStruct((NC, n, 8), jnp.float32),
        mesh=_mesh(),
        scratch_types=[
            pltpu.VMEM((ch, b), jnp.int32),
            pltpu.VMEM((b, 8), jnp.float32),
            pltpu.VMEM_SHARED((n, 8), jnp.float32),
            pltpu.SemaphoreType.DMA,
        ],
        compiler_params=pltpu.CompilerParams(use_tc_tiling_on_sc=False),
    )
    def deg_kernel(dst_hbm, ones_hbm, zeros_hbm, out_hbm, idx_v, ones_v, table,
                   ssem):
        c = lax.axis_index("c")
        s = lax.axis_index("s")
        w = c * NS + s
        pltpu.sync_copy(zeros_hbm, table.at[pl.ds(s * rows, rows)])
        pltpu.sync_copy(ones_hbm, ones_v)
        pltpu.sync_copy(dst_hbm.at[w], idx_v)
        plsc.subcore_barrier()

        def body(t, carry):
            base = t * ring
            ds_ = [pltpu.async_copy(ones_v, table.at[idx_v.at[base + r]],
                                    ssem, add=True)
                   for r in range(ring)]
            for d in ds_:
                d.wait()
            return carry

        lax.fori_loop(0, ch // ring, body, 0)
        plsc.subcore_barrier()
        pltpu.sync_copy(table.at[pl.ds(s * rows, rows)],
                        out_hbm.at[c, pl.ds(s * rows, rows)])

    return deg_kernel


def _make_sc_scatter(n, hw, ch, b, ring):
    rows = n // NS

    @functools.partial(
        pl.kernel,
        out_type=jax.ShapeDtypeStruct((NC, n, hw), jnp.float32),
        mesh=_mesh(),
        scratch_types=[
            pltpu.VMEM((ch, b), jnp.int32),
            pltpu.VMEM((ch, b), jnp.int32),
            pltpu.VMEM((ring, b, hw), jnp.float32),
            pltpu.VMEM_SHARED((n, hw), jnp.float32),
        ] + [pltpu.SemaphoreType.DMA] * 9,
        compiler_params=pltpu.CompilerParams(use_tc_tiling_on_sc=False),
    )
    def scat_kernel(src_hbm, dst_hbm, xs_hbm, zeros_hbm, out_hbm,
                    si, di, rowbuf, table, *sems):
        gsems, ssem = sems[:8], sems[8]
        c = lax.axis_index("c")
        s = lax.axis_index("s")
        w = c * NS + s
        pltpu.sync_copy(zeros_hbm, table.at[pl.ds(s * rows, rows)])
        pltpu.sync_copy(src_hbm.at[w], si)
        pltpu.sync_copy(dst_hbm.at[w], di)
        plsc.subcore_barrier()

        def body(t, carry):
            base = t * ring
            gds = [pltpu.async_copy(xs_hbm.at[si.at[base + r]], rowbuf.at[r],
                                    gsems[r])
                   for r in range(ring)]
            for r in range(ring):
                gds[r].wait()
                pltpu.sync_copy(rowbuf.at[r], table.at[di.at[base + r]],
                                add=True)
            return carry

        lax.fori_loop(0, ch // ring, body, 0)
        plsc.subcore_barrier()
        pltpu.sync_copy(table.at[pl.ds(s * rows, rows)],
                        out_hbm.at[c, pl.ds(s * rows, rows)])

    return scat_kernel


def _make_sc_edge(e, h2, ch, b, ew, ring, eoff):
    @functools.partial(
        pl.kernel,
        out_type=jax.ShapeDtypeStruct((e, h2), jnp.float32),
        mesh=_mesh(),
        scratch_types=[
            pltpu.VMEM((ew,), jnp.int32),
            pltpu.VMEM((ew,), jnp.int32),
            pltpu.VMEM((ring, b, h2), jnp.float32),
            pltpu.SemaphoreType.DMA((8,)),
            pltpu.SemaphoreType.DMA((8,)),
            pltpu.SemaphoreType.DMA,
        ],
    )
    def edge_kernel(src_hbm, dst_hbm, p_hbm, q_hbm, out_hbm, si, di, tbuf,
                    gsem, qsem, wsem):
        c = lax.axis_index("c")
        s = lax.axis_index("s")
        w = c * NS + s
        woff = eoff + w * ew
        pltpu.sync_copy(src_hbm.at[pl.ds(woff, ew)], si)
        pltpu.sync_copy(dst_hbm.at[pl.ds(woff, ew)], di)

        def body(t, carry):
            base = t * ring
            pds = [pltpu.async_copy(
                       p_hbm.at[si.at[pl.ds((base + r) * b, b)]], tbuf.at[r],
                       gsem.at[r])
                   for r in range(ring)]
            qds = []
            for r in range(ring):
                pds[r].wait()
                qds.append(pltpu.async_copy(
                    q_hbm.at[di.at[pl.ds((base + r) * b, b)]],
                    tbuf.at[r], qsem.at[r], add=True))
            wds = []
            for r in range(ring):
                qds[r].wait()
                j = base + r
                wds.append(pltpu.async_copy(
                    tbuf.at[r], out_hbm.at[pl.ds(w * ew + j * b, b)], wsem))
            for d in wds:
                d.wait()
            return carry

        lax.fori_loop(0, ch // ring, body, 0)

    return edge_kernel


def _tc_pre(x, w1, degp):
    n, d = x.shape
    h = w1.shape[1]

    def body(x_ref, w1_ref, degp_ref, xs_ref, dinv_ref):
        deg = degp_ref[0, :, 0:1] + degp_ref[1, :, 0:1] + 1.0
        dinv = 1.0 / jnp.sqrt(deg)
        xw = jnp.dot(x_ref[...], w1_ref[...], preferred_element_type=jnp.float32)
        xs_ref[...] = xw * dinv
        dinv_ref[...] = dinv

    return pl.pallas_call(
        body,
        out_shape=[
            jax.ShapeDtypeStruct((n, h), jnp.float32),
            jax.ShapeDtypeStruct((n, 1), jnp.float32),
        ],
    )(x, w1, degp)


def _tc_mid(acc1, xs1, dinv, b1r, w2):
    n = xs1.shape[0]
    h = w2.shape[0]

    def body(acc_ref, xs_ref, dinv_ref, b1_ref, w2_ref, xs2_ref):
        dinv = dinv_ref[...]
        t = acc_ref[0] + acc_ref[1] + xs_ref[...]
        h1 = jnp.maximum(dinv * t + b1_ref[...], 0.0)
        xw2 = jnp.dot(h1, w2_ref[...], preferred_element_type=jnp.float32)
        xs2_ref[...] = xw2 * dinv

    return pl.pallas_call(
        body,
        out_shape=jax.ShapeDtypeStruct((n, h), jnp.float32),
    )(acc1, xs1, dinv, b1r, w2)


def _tc_post(acc2, xs2, dinv, b2r, wtop, wbot, btop):
    n = xs2.shape[0]
    h = wtop.shape[0]
    h2 = wtop.shape[1]

    def body(acc_ref, xs_ref, dinv_ref, b2_ref, wt_ref, wb_ref, bt_ref,
             p_ref, q_ref):
        dinv = dinv_ref[...]
        t = acc_ref[0] + acc_ref[1] + xs_ref[...]
        hh = dinv * t + b2_ref[...]
        p_ref[...] = jnp.dot(hh, wt_ref[...],
                             preferred_element_type=jnp.float32) + bt_ref[...]
        q_ref[...] = jnp.dot(hh, wb_ref[...],
                             preferred_element_type=jnp.float32)

    return pl.pallas_call(
        body,
        out_shape=[
            jax.ShapeDtypeStruct((n, h2), jnp.float32),
            jax.ShapeDtypeStruct((n, h2), jnp.float32),
        ],
    )(acc2, xs2, dinv, b2r, wtop, wbot, btop)


def _tc_final(t, wcat, bcat, rb):
    e, h2 = t.shape
    co = wcat.shape[1]

    def body(t_ref, w_ref, b_ref, lo_ref, w_out_ref):
        i = pl.program_id(0)
        u = jnp.maximum(t_ref[...], 0.0)
        cat = jnp.dot(u, w_ref[...], preferred_element_type=jnp.float32)
        cat = cat + b_ref[...]
        lo_ref[...] = cat[:, :10]
        z = cat[:, 10]
        w_out_ref[pl.ds(i * rb, rb)] = 1.0 / (1.0 + jnp.exp(-z))

    grid = (e // rb,)
    return pl.pallas_call(
        body,
        grid=grid,
        in_specs=[
            pl.BlockSpec((rb, h2), lambda i: (i, 0)),
            pl.BlockSpec((h2, co), lambda i: (0, 0)),
            pl.BlockSpec((1, co), lambda i: (0, 0)),
        ],
        out_specs=[
            pl.BlockSpec((rb, 10), lambda i: (i, 0)),
            pl.BlockSpec((e,), lambda i: (0,)),
        ],
        out_shape=[
            jax.ShapeDtypeStruct((e, 10), jnp.float32),
            jax.ShapeDtypeStruct((e,), jnp.float32),
        ],
    )(t, wcat, bcat)


def kernel(fact_embeddings, edge_index, W1, b1, W2, b2, Wc1, bc1, Wc2, bc2,
           Ws1, bs1, Ws2, bs2):
    x = fact_embeddings
    n, d = x.shape
    h = W1.shape[1]
    e = edge_index.shape[1]
    h2 = 2 * h

    npad = -(-n // (NS * 8)) * (NS * 8)
    xp = jnp.pad(x, ((0, npad - n), (0, 0)))

    ew = e // NW
    bc = 100
    chc = ew // bc
    be = 80
    che = ew // be

    srcC = edge_index[0].reshape(NW, chc, bc)
    dstC = edge_index[1].reshape(NW, chc, bc)
    ea = (e * 3) // 5
    eb = e - ea
    cha = (ea // NW) // be
    chb = (eb // NW) // be

    ones_bw = jnp.ones((bc, 8), jnp.float32)
    zeros_n8 = jnp.zeros((npad // NS, 8), jnp.float32)
    zeros_nh = jnp.zeros((npad // NS, h), jnp.float32)

    b1r = b1.reshape(1, h)
    b2r = b2.reshape(1, h)
    wtop = jnp.concatenate([Wc1[:h], Ws1[:h]], axis=1)
    wbot = jnp.concatenate([Wc1[h:], Ws1[h:]], axis=1)
    btop = jnp.concatenate([bc1, bs1]).reshape(1, h2)
    nc2 = Wc2.shape[1]
    z1 = jnp.zeros((h, 16 - nc2 - 1), jnp.float32)
    z2 = jnp.zeros((h, nc2), jnp.float32)
    wcat = jnp.concatenate([
        jnp.concatenate([Wc2, Ws2 * 0.0, z1], axis=1),
        jnp.concatenate([z2, Ws2, z1], axis=1),
    ], axis=0)
    bcat = jnp.concatenate([bc2, bs2, jnp.zeros((16 - nc2 - 1,), jnp.float32)])
    bcat = bcat.reshape(1, 16)

    degp = _make_sc_degree(npad, chc, bc, 4)(dstC, ones_bw, zeros_n8)
    xs1, dinv = _tc_pre(xp, W1, degp)
    acc1 = _make_sc_scatter(npad, h, chc, bc, 5)(srcC, dstC, xs1, zeros_nh)
    xs2 = _tc_mid(acc1, xs1, dinv, b1r, W2)
    acc2 = _make_sc_scatter(npad, h, chc, bc, 5)(srcC, dstC, xs2, zeros_nh)
    p, q = _tc_post(acc2, xs2, dinv, b2r, wtop, wbot, btop)
    src1d = edge_index[0]
    dst1d = edge_index[1]
    t1 = _make_sc_edge(ea, h2, cha, be, ea // NW, 5, 0)(src1d, dst1d, p, q)
    t2 = _make_sc_edge(eb, h2, chb, be, eb // NW, 5, ea)(src1d, dst1d, p, q)
    lo1, wr1 = _tc_final(t1, wcat, bcat, rb=6400)
    lo2, wr2 = _tc_final(t2, wcat, bcat, rb=6400)
    logits = jnp.concatenate([lo1, lo2], axis=0)
    w2 = jnp.concatenate([wr1, wr2], axis=0)
    return (logits, w2)


# speedup vs baseline: 1.0104x; 1.0104x over previous
import functools

import jax
import jax.numpy as jnp
from jax import lax
from jax.experimental import pallas as pl
from jax.experimental.pallas import tpu as pltpu
from jax.experimental.pallas import tpu_sc as plsc

NC = 2
NS = 16
NW = NC * NS


def _mesh():
    return plsc.VectorSubcoreMesh(
        core_axis_name="c", subcore_axis_name="s", num_cores=NC, num_subcores=NS
    )


def _make_sc_degree(n, ch, b, ring):
    rows = n // NS

    @functools.partial(
        pl.kernel,
        out_type=jax.ShapeDtypeStruct((NC, n, 8), jnp.float32),
        mesh=_mesh(),
        scratch_types=[
            pltpu.VMEM((ch, b), jnp.int32),
            pltpu.VMEM((b, 8), jnp.float32),
            pltpu.VMEM_SHARED((n, 8), jnp.float32),
            pltpu.SemaphoreType.DMA,
        ],
        compiler_params=pltpu.CompilerParams(use_tc_tiling_on_sc=False),
    )
    def deg_kernel(dst_hbm, ones_hbm, zeros_hbm, out_hbm, idx_v, ones_v, table,
                   ssem):
        c = lax.axis_index("c")
        s = lax.axis_index("s")
        w = c * NS + s
        pltpu.sync_copy(zeros_hbm, table.at[pl.ds(s * rows, rows)])
        pltpu.sync_copy(ones_hbm, ones_v)
        pltpu.sync_copy(dst_hbm.at[w], idx_v)
        plsc.subcore_barrier()

        def body(t, carry):
            base = t * ring
            ds_ = [pltpu.async_copy(ones_v, table.at[idx_v.at[base + r]],
                                    ssem, add=True)
                   for r in range(ring)]
            for d in ds_:
                d.wait()
            return carry

        lax.fori_loop(0, ch // ring, body, 0)
        plsc.subcore_barrier()
        pltpu.sync_copy(table.at[pl.ds(s * rows, rows)],
                        out_hbm.at[c, pl.ds(s * rows, rows)])

    return deg_kernel


def _make_sc_scatter(n, hw, ch, b, ring):
    rows = n // NS

    @functools.partial(
        pl.kernel,
        out_type=jax.ShapeDtypeStruct((NC, n, hw), jnp.float32),
        mesh=_mesh(),
        scratch_types=[
            pltpu.VMEM((ch, b), jnp.int32),
            pltpu.VMEM((ch, b), jnp.int32),
            pltpu.VMEM((ring, b, hw), jnp.float32),
            pltpu.VMEM_SHARED((n, hw), jnp.float32),
        ] + [pltpu.SemaphoreType.DMA] * 9,
        compiler_params=pltpu.CompilerParams(use_tc_tiling_on_sc=False),
    )
    def scat_kernel(src_hbm, dst_hbm, xs_hbm, zeros_hbm, out_hbm,
                    si, di, rowbuf, table, *sems):
        gsems, ssem = sems[:8], sems[8]
        c = lax.axis_index("c")
        s = lax.axis_index("s")
        w = c * NS + s
        pltpu.sync_copy(zeros_hbm, table.at[pl.ds(s * rows, rows)])
        pltpu.sync_copy(src_hbm.at[w], si)
        pltpu.sync_copy(dst_hbm.at[w], di)
        plsc.subcore_barrier()

        def body(t, carry):
            base = t * ring
            gds = [pltpu.async_copy(xs_hbm.at[si.at[base + r]], rowbuf.at[r],
                                    gsems[r])
                   for r in range(ring)]
            for r in range(ring):
                gds[r].wait()
                pltpu.sync_copy(rowbuf.at[r], table.at[di.at[base + r]],
                                add=True)
            return carry

        lax.fori_loop(0, ch // ring, body, 0)
        plsc.subcore_barrier()
        pltpu.sync_copy(table.at[pl.ds(s * rows, rows)],
                        out_hbm.at[c, pl.ds(s * rows, rows)])

    return scat_kernel


def _make_sc_edge(e, h2, ch, b, ew, ring):
    @functools.partial(
        pl.kernel,
        out_type=jax.ShapeDtypeStruct((e, h2), jnp.float32),
        mesh=_mesh(),
        scratch_types=[
            pltpu.VMEM((ch, b), jnp.int32),
            pltpu.VMEM((ch, b), jnp.int32),
            pltpu.VMEM((ring, b, h2), jnp.float32),
            pltpu.SemaphoreType.DMA((8,)),
            pltpu.SemaphoreType.DMA((8,)),
            pltpu.SemaphoreType.DMA,
        ],
    )
    def edge_kernel(src_hbm, dst_hbm, p_hbm, q_hbm, out_hbm, si, di, tbuf,
                    gsem, qsem, wsem):
        c = lax.axis_index("c")
        s = lax.axis_index("s")
        w = c * NS + s
        pltpu.sync_copy(src_hbm.at[w], si)
        pltpu.sync_copy(dst_hbm.at[w], di)

        def body(t, carry):
            base = t * ring
            pds = [pltpu.async_copy(p_hbm.at[si.at[base + r]], tbuf.at[r],
                                    gsem.at[r])
                   for r in range(ring)]
            qds = []
            for r in range(ring):
                pds[r].wait()
                qds.append(pltpu.async_copy(q_hbm.at[di.at[base + r]],
                                            tbuf.at[r], qsem.at[r], add=True))
            wds = []
            for r in range(ring):
                qds[r].wait()
                j = base + r
                wds.append(pltpu.async_copy(
                    tbuf.at[r], out_hbm.at[pl.ds(w * ew + j * b, b)], wsem))
            for d in wds:
                d.wait()
            return carry

        lax.fori_loop(0, ch // ring, body, 0)

    return edge_kernel


def _tc_pre(x, w1, degp):
    n, d = x.shape
    h = w1.shape[1]

    def body(x_ref, w1_ref, degp_ref, xs_ref, dinv_ref):
        deg = degp_ref[0, :, 0:1] + degp_ref[1, :, 0:1] + 1.0
        dinv = 1.0 / jnp.sqrt(deg)
        xw = jnp.dot(x_ref[...], w1_ref[...], preferred_element_type=jnp.float32)
        xs_ref[...] = xw * dinv
        dinv_ref[...] = dinv

    return pl.pallas_call(
        body,
        out_shape=[
            jax.ShapeDtypeStruct((n, h), jnp.float32),
            jax.ShapeDtypeStruct((n, 1), jnp.float32),
        ],
    )(x, w1, degp)


def _tc_mid(acc1, xs1, dinv, b1r, w2):
    n = xs1.shape[0]
    h = w2.shape[0]

    def body(acc_ref, xs_ref, dinv_ref, b1_ref, w2_ref, xs2_ref):
        dinv = dinv_ref[...]
        t = acc_ref[0] + acc_ref[1] + xs_ref[...]
        h1 = jnp.maximum(dinv * t + b1_ref[...], 0.0)
        xw2 = jnp.dot(h1, w2_ref[...], preferred_element_type=jnp.float32)
        xs2_ref[...] = xw2 * dinv

    return pl.pallas_call(
        body,
        out_shape=jax.ShapeDtypeStruct((n, h), jnp.float32),
    )(acc1, xs1, dinv, b1r, w2)


def _tc_post(acc2, xs2, dinv, b2r, wtop, wbot, btop):
    n = xs2.shape[0]
    h = wtop.shape[0]
    h2 = wtop.shape[1]

    def body(acc_ref, xs_ref, dinv_ref, b2_ref, wt_ref, wb_ref, bt_ref,
             p_ref, q_ref):
        dinv = dinv_ref[...]
        t = acc_ref[0] + acc_ref[1] + xs_ref[...]
        hh = dinv * t + b2_ref[...]
        p_ref[...] = jnp.dot(hh, wt_ref[...],
                             preferred_element_type=jnp.float32) + bt_ref[...]
        q_ref[...] = jnp.dot(hh, wb_ref[...],
                             preferred_element_type=jnp.float32)

    return pl.pallas_call(
        body,
        out_shape=[
            jax.ShapeDtypeStruct((n, h2), jnp.float32),
            jax.ShapeDtypeStruct((n, h2), jnp.float32),
        ],
    )(acc2, xs2, dinv, b2r, wtop, wbot, btop)


def _tc_final(t, wcat, bcat, rb):
    e, h2 = t.shape
    co = wcat.shape[1]

    def body(t_ref, w_ref, b_ref, lo_ref, w_out_ref):
        i = pl.program_id(0)
        u = jnp.maximum(t_ref[...], 0.0)
        cat = jnp.dot(u, w_ref[...], preferred_element_type=jnp.float32)
        cat = cat + b_ref[...]
        lo_ref[...] = cat[:, :10]
        z = cat[:, 10]
        w_out_ref[pl.ds(i * rb, rb)] = 1.0 / (1.0 + jnp.exp(-z))

    grid = (e // rb,)
    return pl.pallas_call(
        body,
        grid=grid,
        in_specs=[
            pl.BlockSpec((rb, h2), lambda i: (i, 0)),
            pl.BlockSpec((h2, co), lambda i: (0, 0)),
            pl.BlockSpec((1, co), lambda i: (0, 0)),
        ],
        out_specs=[
            pl.BlockSpec((rb, 10), lambda i: (i, 0)),
            pl.BlockSpec((e,), lambda i: (0,)),
        ],
        out_shape=[
            jax.ShapeDtypeStruct((e, 10), jnp.float32),
            jax.ShapeDtypeStruct((e,), jnp.float32),
        ],
    )(t, wcat, bcat)


def kernel(fact_embeddings, edge_index, W1, b1, W2, b2, Wc1, bc1, Wc2, bc2,
           Ws1, bs1, Ws2, bs2):
    x = fact_embeddings
    n, d = x.shape
    h = W1.shape[1]
    e = edge_index.shape[1]
    h2 = 2 * h

    npad = -(-n // (NS * 8)) * (NS * 8)
    xp = jnp.pad(x, ((0, npad - n), (0, 0)))

    ew = e // NW
    bc = 100
    chc = ew // bc
    be = 80
    che = ew // be

    srcC = edge_index[0].reshape(NW, chc, bc)
    dstC = edge_index[1].reshape(NW, chc, bc)
    ea = (e * 3) // 5
    eb = e - ea
    cha = (ea // NW) // be
    chb = (eb // NW) // be
    srcE1 = edge_index[0, :ea].reshape(NW, cha, be)
    dstE1 = edge_index[1, :ea].reshape(NW, cha, be)
    srcE2 = edge_index[0, ea:].reshape(NW, chb, be)
    dstE2 = edge_index[1, ea:].reshape(NW, chb, be)

    ones_bw = jnp.ones((bc, 8), jnp.float32)
    zeros_n8 = jnp.zeros((npad // NS, 8), jnp.float32)
    zeros_nh = jnp.zeros((npad // NS, h), jnp.float32)

    b1r = b1.reshape(1, h)
    b2r = b2.reshape(1, h)
    wtop = jnp.concatenate([Wc1[:h], Ws1[:h]], axis=1)
    wbot = jnp.concatenate([Wc1[h:], Ws1[h:]], axis=1)
    btop = jnp.concatenate([bc1, bs1]).reshape(1, h2)
    nc2 = Wc2.shape[1]
    z1 = jnp.zeros((h, 16 - nc2 - 1), jnp.float32)
    z2 = jnp.zeros((h, nc2), jnp.float32)
    wcat = jnp.concatenate([
        jnp.concatenate([Wc2, Ws2 * 0.0, z1], axis=1),
        jnp.concatenate([z2, Ws2, z1], axis=1),
    ], axis=0)
    bcat = jnp.concatenate([bc2, bs2, jnp.zeros((16 - nc2 - 1,), jnp.float32)])
    bcat = bcat.reshape(1, 16)

    degp = _make_sc_degree(npad, chc, bc, 4)(dstC, ones_bw, zeros_n8)
    xs1, dinv = _tc_pre(xp, W1, degp)
    acc1 = _make_sc_scatter(npad, h, chc, bc, 5)(srcC, dstC, xs1, zeros_nh)
    xs2 = _tc_mid(acc1, xs1, dinv, b1r, W2)
    acc2 = _make_sc_scatter(npad, h, chc, bc, 5)(srcC, dstC, xs2, zeros_nh)
    p, q = _tc_post(acc2, xs2, dinv, b2r, wtop, wbot, btop)
    t1 = _make_sc_edge(ea, h2, cha, be, ea // NW, 5)(srcE1, dstE1, p, q)
    t2 = _make_sc_edge(eb, h2, chb, be, eb // NW, 5)(srcE2, dstE2, p, q)
    lo1, wr1 = _tc_final(t1, wcat, bcat, rb=6400)
    lo2, wr2 = _tc_final(t2, wcat, bcat, rb=6400)
    logits = jnp.concatenate([lo1, lo2], axis=0)
    w2 = jnp.concatenate([wr1, wr2], axis=0)
    return (logits, w2)
